# baseline (device time: 109254 ns/iter reference)
import jax
import jax.numpy as jnp
from jax import lax
from jax.experimental import pallas as pl
from jax.experimental.pallas import tpu as pltpu

def _chunk_schedule(half):
    ramp_in = [16, 16, 32, 64]
    ramp_out = [64, 32, 16, 16]
    mid = half - sum(ramp_in) - sum(ramp_out)
    assert mid % 128 == 0
    rows = ramp_in + [128] * (mid // 128) + ramp_out
    offs = []
    o = 0
    for r in rows:
        offs.append(o)
        o += r
    return list(zip(offs, rows))


def kernel(x):
    m, n = x.shape
    half = m // 2
    chunks = _chunk_schedule(half)
    C = len(chunks)

    def body(x_hbm, out_hbm, xv, comm, sums, in_sems, out_sems,
             x_send_sems, x_recv_sems, y_send_sems, y_recv_sems):
        my_x = lax.axis_index("x")
        my_y = lax.axis_index("y")
        x_nbr = (1 - my_x, my_y)
        y_nbr = (my_x, 1 - my_y)
        my_off = my_y * half
        other_off = (1 - my_y) * half

        barrier = pltpu.get_barrier_semaphore()
        for nbr in (x_nbr, y_nbr):
            pl.semaphore_signal(
                barrier, inc=1, device_id=nbr,
                device_id_type=pl.DeviceIdType.MESH,
            )
        pl.semaphore_wait(barrier, 2)

        local_ins = []
        x_rdmas = []
        for c, (off, rows) in enumerate(chunks):
            src = x_hbm.at[pl.ds(my_off + off, rows), :]
            cp = pltpu.make_async_copy(
                src, xv.at[pl.ds(off, rows), :], in_sems.at[c]
            )
            cp.start()
            local_ins.append(cp)
            rd = pltpu.make_async_remote_copy(
                src_ref=src,
                dst_ref=comm.at[pl.ds(off, rows), :],
                send_sem=x_send_sems.at[c],
                recv_sem=x_recv_sems.at[c],
                device_id=x_nbr,
                device_id_type=pl.DeviceIdType.MESH,
            )
            rd.start()
            x_rdmas.append(rd)

        y_rdmas = []
        local_outs = []
        for c, (off, rows) in enumerate(chunks):
            x_rdmas[c].wait_recv()
            local_ins[c].wait()
            cs = pl.ds(off, rows)
            sums[cs, :] = xv[cs, :] + comm[cs, :]
            rd = pltpu.make_async_remote_copy(
                src_ref=sums.at[cs, :],
                dst_ref=out_hbm.at[pl.ds(my_off + off, rows), :],
                send_sem=y_send_sems.at[c],
                recv_sem=y_recv_sems.at[c],
                device_id=y_nbr,
                device_id_type=pl.DeviceIdType.MESH,
            )
            rd.start()
            y_rdmas.append(rd)
            cp = pltpu.make_async_copy(
                sums.at[cs, :],
                out_hbm.at[pl.ds(my_off + off, rows), :],
                out_sems.at[c],
            )
            cp.start()
            local_outs.append(cp)

        for c, (off, rows) in enumerate(chunks):
            recv = pltpu.make_async_remote_copy(
                src_ref=sums.at[pl.ds(off, rows), :],
                dst_ref=out_hbm.at[pl.ds(other_off + off, rows), :],
                send_sem=x_send_sems.at[c],
                recv_sem=y_recv_sems.at[c],
                device_id=y_nbr,
                device_id_type=pl.DeviceIdType.MESH,
            )
            recv.wait_recv()
            local_outs[c].wait()
            x_rdmas[c].wait_send()
            y_rdmas[c].wait_send()

    return pl.pallas_call(
        body,
        out_shape=jax.ShapeDtypeStruct((m, n), x.dtype),
        in_specs=[pl.BlockSpec(memory_space=pl.ANY)],
        out_specs=pl.BlockSpec(memory_space=pl.ANY),
        scratch_shapes=[
            pltpu.VMEM((half, n), x.dtype),
            pltpu.VMEM((half, n), x.dtype),
            pltpu.VMEM((half, n), x.dtype),
            pltpu.SemaphoreType.DMA((C,)),
            pltpu.SemaphoreType.DMA((C,)),
            pltpu.SemaphoreType.DMA((C,)),
            pltpu.SemaphoreType.DMA((C,)),
            pltpu.SemaphoreType.DMA((C,)),
            pltpu.SemaphoreType.DMA((C,)),
        ],
        compiler_params=pltpu.CompilerParams(collective_id=0),
    )(x)


# device time: 107150 ns/iter; 1.0196x vs baseline; 1.0196x over previous
import jax
import jax.numpy as jnp
from jax import lax
from jax.experimental import pallas as pl
from jax.experimental.pallas import tpu as pltpu

def _chunk_schedule(half):
    ramp_in = [16, 16, 32]
    ramp_out = [32, 16, 16]
    mid = half - sum(ramp_in) - sum(ramp_out)
    assert mid % 64 == 0
    rows = ramp_in + [64] * (mid // 64) + ramp_out
    offs = []
    o = 0
    for r in rows:
        offs.append(o)
        o += r
    return list(zip(offs, rows))


def kernel(x):
    m, n = x.shape
    half = m // 2
    chunks = _chunk_schedule(half)
    C = len(chunks)

    def body(x_hbm, out_hbm, xv, comm, sums, in_sems, out_sems,
             x_send_sems, x_recv_sems, y_send_sems, y_recv_sems):
        my_x = lax.axis_index("x")
        my_y = lax.axis_index("y")
        x_nbr = (1 - my_x, my_y)
        y_nbr = (my_x, 1 - my_y)
        my_off = my_y * half
        other_off = (1 - my_y) * half

        barrier = pltpu.get_barrier_semaphore()
        for nbr in (x_nbr, y_nbr):
            pl.semaphore_signal(
                barrier, inc=1, device_id=nbr,
                device_id_type=pl.DeviceIdType.MESH,
            )
        pl.semaphore_wait(barrier, 2)

        local_ins = []
        x_rdmas = []
        for c, (off, rows) in enumerate(chunks):
            src = x_hbm.at[pl.ds(my_off + off, rows), :]
            cp = pltpu.make_async_copy(
                src, xv.at[pl.ds(off, rows), :], in_sems.at[c]
            )
            cp.start()
            local_ins.append(cp)
            rd = pltpu.make_async_remote_copy(
                src_ref=src,
                dst_ref=comm.at[pl.ds(off, rows), :],
                send_sem=x_send_sems.at[c],
                recv_sem=x_recv_sems.at[c],
                device_id=x_nbr,
                device_id_type=pl.DeviceIdType.MESH,
            )
            rd.start()
            x_rdmas.append(rd)

        y_rdmas = []
        local_outs = []
        for c, (off, rows) in enumerate(chunks):
            x_rdmas[c].wait_recv()
            local_ins[c].wait()
            cs = pl.ds(off, rows)
            sums[cs, :] = xv[cs, :] + comm[cs, :]
            rd = pltpu.make_async_remote_copy(
                src_ref=sums.at[cs, :],
                dst_ref=out_hbm.at[pl.ds(my_off + off, rows), :],
                send_sem=y_send_sems.at[c],
                recv_sem=y_recv_sems.at[c],
                device_id=y_nbr,
                device_id_type=pl.DeviceIdType.MESH,
            )
            rd.start()
            y_rdmas.append(rd)
            cp = pltpu.make_async_copy(
                sums.at[cs, :],
                out_hbm.at[pl.ds(my_off + off, rows), :],
                out_sems.at[c],
            )
            cp.start()
            local_outs.append(cp)

        for c, (off, rows) in enumerate(chunks):
            recv = pltpu.make_async_remote_copy(
                src_ref=sums.at[pl.ds(off, rows), :],
                dst_ref=out_hbm.at[pl.ds(other_off + off, rows), :],
                send_sem=x_send_sems.at[c],
                recv_sem=y_recv_sems.at[c],
                device_id=y_nbr,
                device_id_type=pl.DeviceIdType.MESH,
            )
            recv.wait_recv()
            local_outs[c].wait()
            x_rdmas[c].wait_send()
            y_rdmas[c].wait_send()

    return pl.pallas_call(
        body,
        out_shape=jax.ShapeDtypeStruct((m, n), x.dtype),
        in_specs=[pl.BlockSpec(memory_space=pl.ANY)],
        out_specs=pl.BlockSpec(memory_space=pl.ANY),
        scratch_shapes=[
            pltpu.VMEM((half, n), x.dtype),
            pltpu.VMEM((half, n), x.dtype),
            pltpu.VMEM((half, n), x.dtype),
            pltpu.SemaphoreType.DMA((C,)),
            pltpu.SemaphoreType.DMA((C,)),
            pltpu.SemaphoreType.DMA((C,)),
            pltpu.SemaphoreType.DMA((C,)),
            pltpu.SemaphoreType.DMA((C,)),
            pltpu.SemaphoreType.DMA((C,)),
        ],
        compiler_params=pltpu.CompilerParams(collective_id=0),
    )(x)
